# Initial kernel scaffold; baseline (speedup 1.0000x reference)
import jax
import jax.numpy as jnp
from jax.experimental import pallas as pl


def kernel(x, e, embeddings):
    raise NotImplementedError("write your pallas kernel here")


# speedup vs baseline: 1.6432x; 1.6432x over previous
import functools

import jax
import jax.numpy as jnp
from jax.experimental import pallas as pl
from jax.experimental.pallas import tpu as pltpu

N_NODES = 100000
EMB_DIM = 128
NUM_EMB = 512
COMMITMENT_COST = 0.25

BLOCK = 2000
GRID = N_NODES // BLOCK


def _vq_block(at_ref, e_ref, w_ref, q_ref, loss_ref):
    e = e_ref[...]
    w = w_ref[...]
    at = at_ref[...]

    esq = jnp.sum(e * e, axis=1, keepdims=True)
    wsq = jnp.sum(w * w, axis=1)[None, :]
    m = jax.lax.dot_general(
        e, w, (((1,), (1,)), ((), ())),
        preferred_element_type=jnp.float32)
    d = (esq + wsq) - 2.0 * m

    lo = jnp.where(at == 5, 0, jnp.where(at == 6, 378, jnp.where(at == 7, 434, 489)))
    hi = jnp.where(at == 5, 377, jnp.where(at == 6, 433, jnp.where(at == 7, 488, 511)))
    col = jax.lax.broadcasted_iota(jnp.int32, (BLOCK, NUM_EMB), 1)
    valid = (col >= lo) & (col < hi)
    dm = jnp.where(valid, d, jnp.inf)

    dmin = jnp.min(dm, axis=1, keepdims=True)
    enc = jnp.min(jnp.where(dm == dmin, col, NUM_EMB), axis=1, keepdims=True)

    onehot = (col == enc).astype(jnp.float32)
    q = jax.lax.dot_general(
        onehot, w, (((1,), (0,)), ((), ())),
        preferred_element_type=jnp.float32)

    diff = q - e
    q_ref[...] = e + diff
    loss_ref[...] = jnp.broadcast_to(
        jnp.sum(diff * diff).reshape(1, 1, 1), (1, 1, EMB_DIM))


@jax.jit
def _vq(atom, e, embeddings):
    q, lpart = pl.pallas_call(
        _vq_block,
        grid=(GRID,),
        in_specs=[
            pl.BlockSpec((BLOCK, 1), lambda i: (i, 0)),
            pl.BlockSpec((BLOCK, EMB_DIM), lambda i: (i, 0)),
            pl.BlockSpec((NUM_EMB, EMB_DIM), lambda i: (0, 0)),
        ],
        out_specs=[
            pl.BlockSpec((BLOCK, EMB_DIM), lambda i: (i, 0)),
            pl.BlockSpec((1, 1, EMB_DIM), lambda i: (i, 0, 0)),
        ],
        out_shape=[
            jax.ShapeDtypeStruct((N_NODES, EMB_DIM), jnp.float32),
            jax.ShapeDtypeStruct((GRID, 1, EMB_DIM), jnp.float32),
        ],
        compiler_params=pltpu.CompilerParams(
            dimension_semantics=("parallel",)),
    )(atom, e, embeddings)
    loss = jnp.sum(lpart[:, 0, 0]) * ((1.0 + COMMITMENT_COST) / (N_NODES * EMB_DIM))
    return q, loss


def kernel(x, e, embeddings):
    atom = x[:, 0:1].astype(jnp.int32)
    return _vq(atom, e, embeddings)
